# baseline (speedup 1.0000x reference)
import functools

import jax
import jax.numpy as jnp
from jax import lax
from jax.experimental import pallas as pl
from jax.experimental.pallas import tpu as pltpu
from jax.experimental.pallas import tpu_sc as plsc

H = 64
HH = 32
ROW = 64
C = 80
NC = 2
NS = 16


def _silu(x):
    return x / (1.0 + jnp.exp(-x))


def _stage1_body(pos_ref, wem_ref, bem_ref, we_ref, be_ref,
                 h_ref, sa0_ref, sa1_ref, sb0_ref, sb1_ref):
    pos = pos_ref[...]
    h = jnp.dot(pos, wem_ref[...],
                preferred_element_type=jnp.float32) + bem_ref[...]
    sq = jnp.sum(pos * pos, axis=1, keepdims=True)
    We = we_ref[...]
    A = jnp.dot(h, We[0:H], preferred_element_type=jnp.float32)
    B = jnp.dot(h, We[H:2 * H], preferred_element_type=jnp.float32)
    wd = We[2 * H:2 * H + 1]
    A2 = A + sq * wd
    B2 = B + sq * wd + be_ref[...]
    z = jnp.zeros((pos.shape[0], ROW - 35), jnp.float32)
    P = jnp.concatenate([pos, z], axis=1)
    Pm = jnp.concatenate([-2.0 * pos, z], axis=1)
    h_ref[...] = h
    sa0_ref[...] = jnp.concatenate([A2[:, :HH], P], axis=1)
    sa1_ref[...] = jnp.concatenate([A2[:, HH:], P], axis=1)
    sb0_ref[...] = jnp.concatenate([B2[:, :HH], Pm], axis=1)
    sb1_ref[...] = jnp.concatenate([B2[:, HH:], Pm], axis=1)


def _stage1(pos, W_embed, b_embed, W_e, b_e, interpret=False):
    N = pos.shape[0]
    R = 400 if N % 400 == 0 else N
    grid = N // R
    full = lambda shape: pl.BlockSpec(shape, lambda i: (0, 0))
    row_blk = lambda w: pl.BlockSpec((R, w), lambda i: (i, 0))
    return pl.pallas_call(
        _stage1_body,
        grid=(grid,),
        in_specs=[row_blk(3), full((3, H)), full((1, H)),
                  full((2 * H + 1, H)), full((1, H))],
        out_specs=[row_blk(H), row_blk(ROW), row_blk(ROW),
                   row_blk(ROW), row_blk(ROW)],
        out_shape=[jax.ShapeDtypeStruct((N, H), jnp.float32)] +
                  [jax.ShapeDtypeStruct((N, ROW), jnp.float32)] * 4,
        interpret=interpret,
    )(pos, W_embed, b_embed.reshape(1, H), W_e, b_e.reshape(1, H))


def _sc_edge(ei, SA, SB, wd2, N, E, interpret=False):
    per_tile = E // NS
    nchunks = per_tile // C
    Npad = -(-N // (NS * 8)) * (NS * 8)
    rows_per_tile = Npad // NS
    ZR = max(d for d in range(1, 161) if rows_per_tile % d == 0)
    nzc = rows_per_tile // ZR
    mesh = plsc.VectorSubcoreMesh(core_axis_name="c", subcore_axis_name="s")

    NB = 2

    @functools.partial(
        pl.kernel,
        out_type=jax.ShapeDtypeStruct((NC, Npad, HH), jnp.bfloat16),
        mesh=mesh,
        scratch_types=(
            [pltpu.VMEM((C,), jnp.int32)] * (4 * NB) +
            [pltpu.VMEM((C, ROW), jnp.float32)] * (2 * NB) +
            [pltpu.VMEM((C, HH), jnp.bfloat16)] * NB +
            [pltpu.VMEM((HH,), jnp.float32),
             pltpu.VMEM((ZR, HH), jnp.bfloat16),
             pltpu.VMEM_SHARED((Npad, HH), jnp.bfloat16)] +
            [pltpu.SemaphoreType.DMA] * (3 * NB)
        ),
        compiler_params=pltpu.CompilerParams(use_tc_tiling_on_sc=False,
                                             needs_layout_passes=False),
        interpret=interpret,
    )
    def k(ei_h, sa_h, sb_h, wd_h, out_h,
          src0, src1, dst0, dst1, dstg0, dstg1, dsts0, dsts1,
          a0, a1, b0, b1, m0_v, m1_v,
          wd_v, zb_v, agg_sh, si0, si1, sg0, sg1, ss0, ss1):
        c = lax.axis_index("c")
        s = lax.axis_index("s")
        cN = c * N
        slots = ((src0, dst0, dstg0, a0, b0, m0_v, si0, sg0, ss0, dsts0),
                 (src1, dst1, dstg1, a1, b1, m1_v, si1, sg1, ss1, dsts1))
        pltpu.sync_copy(wd_h.at[c], wd_v)
        wd0 = wd_v[pl.ds(0, 16)]
        wd1 = wd_v[pl.ds(16, 16)]

        z32 = jnp.zeros((HH,), jnp.bfloat16)

        def zrow(i, carry):
            zb_v[i, :] = z32
            return carry

        lax.fori_loop(0, ZR, zrow, 0)
        tile_row0 = s * rows_per_tile

        def zcopy(i, carry):
            pltpu.sync_copy(zb_v, agg_sh.at[pl.ds(tile_row0 + i * ZR, ZR)])
            return carry

        lax.fori_loop(0, nzc, zcopy, 0)
        plsc.subcore_barrier()

        ebase = s * per_tile

        def idx_copies(j, sl):
            src_v, dst_v = sl[0], sl[1]
            sem = sl[6]
            base = pl.multiple_of(ebase + j * C, 8)
            return (pltpu.make_async_copy(ei_h.at[pl.ds(base, C)], src_v, sem),
                    pltpu.make_async_copy(ei_h.at[pl.ds(E + base, C)], dst_v,
                                          sem))

        def gather_copies(sl):
            return (pltpu.make_async_copy(sa_h.at[sl[0]], sl[3], sl[7]),
                    pltpu.make_async_copy(sb_h.at[sl[2]], sl[4], sl[7]))

        def offset_ids(sl):
            src_v, dst_v, dstg_v = sl[0], sl[1], sl[2]
            for j in range(C // 16):
                w = pl.ds(j * 16, 16)
                src_v[w] = src_v[w] + cN
                dstg_v[w] = dst_v[w] + cN

        UE = 8

        def compute(sl):
            a_v, b_v, m_v = sl[3], sl[4], sl[5]

            def ebody(kk, carry):
                e0 = kk * UE
                for u in range(UE):
                    e = e0 + u
                    cp = a_v[e, pl.ds(32, 16)] * b_v[e, pl.ds(32, 16)]
                    cr = cp[0] + cp[1] + cp[2]
                    m0 = a_v[e, pl.ds(0, 16)] + b_v[e, pl.ds(0, 16)] + cr * wd0
                    m1 = a_v[e, pl.ds(16, 16)] + b_v[e, pl.ds(16, 16)] + cr * wd1
                    m_v[e, :] = plsc.pack(_silu(m0), _silu(m1),
                                          format=plsc.PackFormat.INTERLEAVED)
                return carry

            lax.fori_loop(0, C // UE, ebody, 0)

        def scatter_copy(sl):
            return pltpu.make_async_copy(sl[5], agg_sh.at[sl[9]], sl[8])

        for cp in idx_copies(0, slots[0]):
            cp.start()
        if nchunks > 1:
            for cp in idx_copies(1, slots[1]):
                cp.start()
        for cp in idx_copies(0, slots[0]):
            cp.wait()
        offset_ids(slots[0])
        for cp in gather_copies(slots[0]):
            cp.start()

        def step(i, p):
            sl = slots[p]
            sq = slots[1 - p]

            @pl.when(i < nchunks - 1)
            def _():
                for cp in idx_copies(i + 1, sq):
                    cp.wait()
                offset_ids(sq)
                for cp in gather_copies(sq):
                    cp.start()

            for cp in gather_copies(sl):
                cp.wait()

            @pl.when(i >= 2)
            def _():
                scatter_copy(sl).wait()

            dst_v, dsts_v = sl[1], sl[9]
            for j in range(C // 16):
                w = pl.ds(j * 16, 16)
                dsts_v[w] = dst_v[w]

            @pl.when(i < nchunks - 2)
            def _():
                for cp in idx_copies(i + 2, sl):
                    cp.start()

            compute(sl)
            scatter_copy(sl).start(add=True)

        def chunk(i, carry):
            @pl.when(i % 2 == 0)
            def _():
                step(i, 0)

            @pl.when(i % 2 == 1)
            def _():
                step(i, 1)

            return carry

        lax.fori_loop(0, nchunks, chunk, 0)
        if nchunks > 1:
            scatter_copy(slots[1]).wait()
        scatter_copy(slots[0]).wait()
        plsc.subcore_barrier()
        pltpu.sync_copy(agg_sh.at[pl.ds(tile_row0, rows_per_tile)],
                        out_h.at[c, pl.ds(tile_row0, rows_per_tile)])

    return k(ei.reshape(2 * E), SA, SB, wd2)


def _stage3_body(h_ref, agg_ref, wh_ref, bh_ref, wfc_ref, bfc_ref, out_ref):
    hv = h_ref[...]
    agg = agg_ref[...]
    Wh = wh_ref[...]
    t = (jnp.dot(hv, Wh[0:H], preferred_element_type=jnp.float32) +
         jnp.dot(agg, Wh[H:2 * H], preferred_element_type=jnp.float32) +
         bh_ref[...])
    hn = _silu(t)
    out_ref[...] = jnp.dot(hn, wfc_ref[...],
                           preferred_element_type=jnp.float32) + bfc_ref[...]


def _stage3(h, agg, W_h, b_h, W_fc, b_fc, interpret=False):
    N = h.shape[0]
    R = 400 if N % 400 == 0 else N
    grid = N // R
    full = lambda shape: pl.BlockSpec(shape, lambda i: (0, 0))
    row_blk = lambda w: pl.BlockSpec((R, w), lambda i: (i, 0))
    return pl.pallas_call(
        _stage3_body,
        grid=(grid,),
        in_specs=[row_blk(H), row_blk(H), full((2 * H, H)), full((1, H)),
                  full((H, 3)), full((1, 3))],
        out_specs=row_blk(3),
        out_shape=jax.ShapeDtypeStruct((N, 3), jnp.float32),
        interpret=interpret,
    )(h, agg, W_h, b_h.reshape(1, H), W_fc, b_fc.reshape(1, 3))


def kernel(pos, edge_index, W_embed, b_embed, W_e, b_e, W_h, b_h, W_fc, b_fc):
    N = pos.shape[0]
    E = edge_index.shape[1]
    h, sa0, sa1, sb0, sb1 = _stage1(pos, W_embed, b_embed, W_e, b_e)
    SA = jnp.concatenate([sa0, sa1], axis=0)
    SB = jnp.concatenate([sb0, sb1], axis=0)
    wd2 = W_e[2 * H].reshape(NC, HH)
    aggs = _sc_edge(edge_index, SA, SB, wd2, N, E)
    perm = [2 * f if f < 16 else 2 * (f - 16) + 1 for f in range(HH)]
    aggf = aggs[:, :N, :].astype(jnp.float32)[:, :, jnp.array(perm)]
    agg = jnp.concatenate([aggf[0], aggf[1]], axis=1)
    return _stage3(h, agg, W_h, b_h, W_fc, b_fc)


# speedup vs baseline: 1.0279x; 1.0279x over previous
import functools

import jax
import jax.numpy as jnp
from jax import lax
from jax.experimental import pallas as pl
from jax.experimental.pallas import tpu as pltpu
from jax.experimental.pallas import tpu_sc as plsc

H = 64
HH = 32
ROW = 48
C = 80
NC = 2
NS = 16


def _silu(x):
    return x / (1.0 + jnp.exp(-x))


def _stage1_body(pos_ref, wem_ref, bem_ref, we_ref, be_ref,
                 h_ref, sa0_ref, sa1_ref, sb0_ref, sb1_ref):
    pos = pos_ref[...]
    h = jnp.dot(pos, wem_ref[...],
                preferred_element_type=jnp.float32) + bem_ref[...]
    sq = jnp.sum(pos * pos, axis=1, keepdims=True)
    We = we_ref[...]
    A = jnp.dot(h, We[0:H], preferred_element_type=jnp.float32)
    B = jnp.dot(h, We[H:2 * H], preferred_element_type=jnp.float32)
    wd = We[2 * H:2 * H + 1]
    A2 = A + sq * wd
    B2 = B + sq * wd + be_ref[...]
    z = jnp.zeros((pos.shape[0], ROW - 35), jnp.float32)
    P = jnp.concatenate([pos, z], axis=1)
    Pm = jnp.concatenate([-2.0 * pos, z], axis=1)
    h_ref[...] = h
    sa0_ref[...] = jnp.concatenate([A2[:, :HH], P], axis=1)
    sa1_ref[...] = jnp.concatenate([A2[:, HH:], P], axis=1)
    sb0_ref[...] = jnp.concatenate([B2[:, :HH], Pm], axis=1)
    sb1_ref[...] = jnp.concatenate([B2[:, HH:], Pm], axis=1)


def _stage1(pos, W_embed, b_embed, W_e, b_e, interpret=False):
    N = pos.shape[0]
    R = 400 if N % 400 == 0 else N
    grid = N // R
    full = lambda shape: pl.BlockSpec(shape, lambda i: (0, 0))
    row_blk = lambda w: pl.BlockSpec((R, w), lambda i: (i, 0))
    return pl.pallas_call(
        _stage1_body,
        grid=(grid,),
        in_specs=[row_blk(3), full((3, H)), full((1, H)),
                  full((2 * H + 1, H)), full((1, H))],
        out_specs=[row_blk(H), row_blk(ROW), row_blk(ROW),
                   row_blk(ROW), row_blk(ROW)],
        out_shape=[jax.ShapeDtypeStruct((N, H), jnp.float32)] +
                  [jax.ShapeDtypeStruct((N, ROW), jnp.float32)] * 4,
        interpret=interpret,
    )(pos, W_embed, b_embed.reshape(1, H), W_e, b_e.reshape(1, H))


def _sc_edge(ei, SA, SB, wd2, N, E, interpret=False):
    per_tile = E // NS
    nchunks = per_tile // C
    Npad = -(-N // (NS * 8)) * (NS * 8)
    rows_per_tile = Npad // NS
    ZR = max(d for d in range(1, 161) if rows_per_tile % d == 0)
    nzc = rows_per_tile // ZR
    mesh = plsc.VectorSubcoreMesh(core_axis_name="c", subcore_axis_name="s")

    NB = 2

    @functools.partial(
        pl.kernel,
        out_type=jax.ShapeDtypeStruct((NC, Npad, HH), jnp.float32),
        mesh=mesh,
        scratch_types=(
            [pltpu.VMEM((C,), jnp.int32)] * (4 * NB) +
            [pltpu.VMEM((C, ROW), jnp.float32)] * (2 * NB) +
            [pltpu.VMEM((C, HH), jnp.float32)] * NB +
            [pltpu.VMEM((HH,), jnp.float32),
             pltpu.VMEM((ZR, HH), jnp.float32),
             pltpu.VMEM_SHARED((Npad, HH), jnp.float32)] +
            [pltpu.SemaphoreType.DMA] * (3 * NB)
        ),
        compiler_params=pltpu.CompilerParams(use_tc_tiling_on_sc=False,
                                             needs_layout_passes=False),
        interpret=interpret,
    )
    def k(ei_h, sa_h, sb_h, wd_h, out_h,
          src0, src1, dst0, dst1, dstg0, dstg1, dsts0, dsts1,
          a0, a1, b0, b1, m0_v, m1_v,
          wd_v, zb_v, agg_sh, si0, si1, sg0, sg1, ss0, ss1):
        c = lax.axis_index("c")
        s = lax.axis_index("s")
        cN = c * N
        slots = ((src0, dst0, dstg0, a0, b0, m0_v, si0, sg0, ss0, dsts0),
                 (src1, dst1, dstg1, a1, b1, m1_v, si1, sg1, ss1, dsts1))
        pltpu.sync_copy(wd_h.at[c], wd_v)
        wd0 = wd_v[pl.ds(0, 16)]
        wd1 = wd_v[pl.ds(16, 16)]

        z16 = jnp.zeros((16,), jnp.float32)

        def zrow(i, carry):
            zb_v[i, pl.ds(0, 16)] = z16
            zb_v[i, pl.ds(16, 16)] = z16
            return carry

        lax.fori_loop(0, ZR, zrow, 0)
        tile_row0 = s * rows_per_tile

        def zcopy(i, carry):
            pltpu.sync_copy(zb_v, agg_sh.at[pl.ds(tile_row0 + i * ZR, ZR)])
            return carry

        lax.fori_loop(0, nzc, zcopy, 0)
        plsc.subcore_barrier()

        ebase = s * per_tile

        def idx_copies(j, sl):
            src_v, dst_v = sl[0], sl[1]
            sem = sl[6]
            base = pl.multiple_of(ebase + j * C, 8)
            return (pltpu.make_async_copy(ei_h.at[pl.ds(base, C)], src_v, sem),
                    pltpu.make_async_copy(ei_h.at[pl.ds(E + base, C)], dst_v,
                                          sem))

        def gather_copies(sl):
            return (pltpu.make_async_copy(sa_h.at[sl[0]], sl[3], sl[7]),
                    pltpu.make_async_copy(sb_h.at[sl[2]], sl[4], sl[7]))

        def offset_ids(sl):
            src_v, dst_v, dstg_v = sl[0], sl[1], sl[2]
            for j in range(C // 16):
                w = pl.ds(j * 16, 16)
                src_v[w] = src_v[w] + cN
                dstg_v[w] = dst_v[w] + cN

        UE = 8

        def compute(sl):
            a_v, b_v, m_v = sl[3], sl[4], sl[5]

            def ebody(kk, carry):
                e0 = kk * UE
                for u in range(UE):
                    e = e0 + u
                    cp = a_v[e, pl.ds(32, 16)] * b_v[e, pl.ds(32, 16)]
                    cr = cp[0] + cp[1] + cp[2]
                    m0 = a_v[e, pl.ds(0, 16)] + b_v[e, pl.ds(0, 16)] + cr * wd0
                    m1 = a_v[e, pl.ds(16, 16)] + b_v[e, pl.ds(16, 16)] + cr * wd1
                    m_v[e, pl.ds(0, 16)] = _silu(m0)
                    m_v[e, pl.ds(16, 16)] = _silu(m1)
                return carry

            lax.fori_loop(0, C // UE, ebody, 0)

        def scatter_copy(sl):
            return pltpu.make_async_copy(sl[5], agg_sh.at[sl[9]], sl[8])

        for cp in idx_copies(0, slots[0]):
            cp.start()
        if nchunks > 1:
            for cp in idx_copies(1, slots[1]):
                cp.start()
        for cp in idx_copies(0, slots[0]):
            cp.wait()
        offset_ids(slots[0])
        for cp in gather_copies(slots[0]):
            cp.start()

        def step(i, p):
            sl = slots[p]
            sq = slots[1 - p]

            @pl.when(i < nchunks - 1)
            def _():
                for cp in idx_copies(i + 1, sq):
                    cp.wait()
                offset_ids(sq)
                for cp in gather_copies(sq):
                    cp.start()

            for cp in gather_copies(sl):
                cp.wait()

            @pl.when(i >= 2)
            def _():
                scatter_copy(sl).wait()

            dst_v, dsts_v = sl[1], sl[9]
            for j in range(C // 16):
                w = pl.ds(j * 16, 16)
                dsts_v[w] = dst_v[w]

            @pl.when(i < nchunks - 2)
            def _():
                for cp in idx_copies(i + 2, sl):
                    cp.start()

            compute(sl)
            scatter_copy(sl).start(add=True)

        def chunk(i, carry):
            @pl.when(i % 2 == 0)
            def _():
                step(i, 0)

            @pl.when(i % 2 == 1)
            def _():
                step(i, 1)

            return carry

        lax.fori_loop(0, nchunks, chunk, 0)
        if nchunks > 1:
            scatter_copy(slots[1]).wait()
        scatter_copy(slots[0]).wait()
        plsc.subcore_barrier()
        pltpu.sync_copy(agg_sh.at[pl.ds(tile_row0, rows_per_tile)],
                        out_h.at[c, pl.ds(tile_row0, rows_per_tile)])

    return k(ei.reshape(2 * E), SA, SB, wd2)


def _stage3_body(h_ref, agg_ref, wh_ref, bh_ref, wfc_ref, bfc_ref, out_ref):
    hv = h_ref[...]
    agg = agg_ref[...]
    Wh = wh_ref[...]
    t = (jnp.dot(hv, Wh[0:H], preferred_element_type=jnp.float32) +
         jnp.dot(agg, Wh[H:2 * H], preferred_element_type=jnp.float32) +
         bh_ref[...])
    hn = _silu(t)
    out_ref[...] = jnp.dot(hn, wfc_ref[...],
                           preferred_element_type=jnp.float32) + bfc_ref[...]


def _stage3(h, agg, W_h, b_h, W_fc, b_fc, interpret=False):
    N = h.shape[0]
    R = 400 if N % 400 == 0 else N
    grid = N // R
    full = lambda shape: pl.BlockSpec(shape, lambda i: (0, 0))
    row_blk = lambda w: pl.BlockSpec((R, w), lambda i: (i, 0))
    return pl.pallas_call(
        _stage3_body,
        grid=(grid,),
        in_specs=[row_blk(H), row_blk(H), full((2 * H, H)), full((1, H)),
                  full((H, 3)), full((1, 3))],
        out_specs=row_blk(3),
        out_shape=jax.ShapeDtypeStruct((N, 3), jnp.float32),
        interpret=interpret,
    )(h, agg, W_h, b_h.reshape(1, H), W_fc, b_fc.reshape(1, 3))


def kernel(pos, edge_index, W_embed, b_embed, W_e, b_e, W_h, b_h, W_fc, b_fc):
    N = pos.shape[0]
    E = edge_index.shape[1]
    h, sa0, sa1, sb0, sb1 = _stage1(pos, W_embed, b_embed, W_e, b_e)
    SA = jnp.concatenate([sa0, sa1], axis=0)
    SB = jnp.concatenate([sb0, sb1], axis=0)
    wd2 = W_e[2 * H].reshape(NC, HH)
    aggs = _sc_edge(edge_index, SA, SB, wd2, N, E)
    agg = jnp.concatenate([aggs[0, :N], aggs[1, :N]], axis=1)
    return _stage3(h, agg, W_h, b_h, W_fc, b_fc)


# speedup vs baseline: 2.1776x; 2.1186x over previous
import functools

import jax
import jax.numpy as jnp
from jax import lax
from jax.experimental import pallas as pl
from jax.experimental.pallas import tpu as pltpu
from jax.experimental.pallas import tpu_sc as plsc

H = 64
HH = 32
ROW = 64
C = 80
NC = 2
NS = 16


def _silu(x):
    return x / (1.0 + jnp.exp(-x))


def _stage1_body(pos_ref, wem_ref, bem_ref, we_ref, be_ref,
                 h_ref, sa0_ref, sa1_ref, sb0_ref, sb1_ref):
    pos = pos_ref[...]
    h = jnp.dot(pos, wem_ref[...],
                preferred_element_type=jnp.float32) + bem_ref[...]
    sq = jnp.sum(pos * pos, axis=1, keepdims=True)
    We = we_ref[...]
    A = jnp.dot(h, We[0:H], preferred_element_type=jnp.float32)
    B = jnp.dot(h, We[H:2 * H], preferred_element_type=jnp.float32)
    wd = We[2 * H:2 * H + 1]
    A2 = A + sq * wd
    B2 = B + sq * wd + be_ref[...]
    h_ref[...] = h
    sa0_ref[...] = A2[:, :HH]
    sa1_ref[...] = A2[:, HH:]
    sb0_ref[...] = B2[:, :HH]
    sb1_ref[...] = B2[:, HH:]


def _stage1(pos, W_embed, b_embed, W_e, b_e, interpret=False):
    N = pos.shape[0]
    R = 400 if N % 400 == 0 else N
    grid = N // R
    full = lambda shape: pl.BlockSpec(shape, lambda i: (0, 0))
    row_blk = lambda w: pl.BlockSpec((R, w), lambda i: (i, 0))
    return pl.pallas_call(
        _stage1_body,
        grid=(grid,),
        in_specs=[row_blk(3), full((3, H)), full((1, H)),
                  full((2 * H + 1, H)), full((1, H))],
        out_specs=[row_blk(H), row_blk(HH), row_blk(HH),
                   row_blk(HH), row_blk(HH)],
        out_shape=[jax.ShapeDtypeStruct((N, H), jnp.float32)] +
                  [jax.ShapeDtypeStruct((N, HH), jnp.float32)] * 4,
        interpret=interpret,
    )(pos, W_embed, b_embed.reshape(1, H), W_e, b_e.reshape(1, H))


def _sc_edge(ei, SA, SB, wd2, N, E, interpret=False):
    per_tile = E // NS
    nchunks = per_tile // C
    Npad = -(-N // (NS * 8)) * (NS * 8)
    rows_per_tile = Npad // NS
    ZR = max(d for d in range(1, 161) if rows_per_tile % d == 0)
    nzc = rows_per_tile // ZR
    mesh = plsc.VectorSubcoreMesh(core_axis_name="c", subcore_axis_name="s")

    NB = 2

    @functools.partial(
        pl.kernel,
        out_type=jax.ShapeDtypeStruct((NC, Npad, HH), jnp.float32),
        mesh=mesh,
        scratch_types=(
            [pltpu.VMEM((C,), jnp.int32)] * (4 * NB) +
            [pltpu.VMEM((C, ROW), jnp.bfloat16)] * (2 * NB) +
            [pltpu.VMEM((C, HH), jnp.float32)] * NB +
            [pltpu.VMEM((HH,), jnp.float32),
             pltpu.VMEM((ZR, HH), jnp.float32),
             pltpu.VMEM_SHARED((Npad, HH), jnp.float32)] +
            [pltpu.SemaphoreType.DMA] * (3 * NB)
        ),
        compiler_params=pltpu.CompilerParams(use_tc_tiling_on_sc=False,
                                             needs_layout_passes=False),
        interpret=interpret,
    )
    def k(ei_h, sa_h, sb_h, wd_h, out_h,
          src0, src1, dst0, dst1, dstg0, dstg1, dsts0, dsts1,
          a0, a1, b0, b1, m0_v, m1_v,
          wd_v, zb_v, agg_sh, si0, si1, sg0, sg1, ss0, ss1):
        c = lax.axis_index("c")
        s = lax.axis_index("s")
        cN = c * N
        slots = ((src0, dst0, dstg0, a0, b0, m0_v, si0, sg0, ss0, dsts0),
                 (src1, dst1, dstg1, a1, b1, m1_v, si1, sg1, ss1, dsts1))
        pltpu.sync_copy(wd_h.at[c], wd_v)
        wd0 = wd_v[pl.ds(0, 16)]
        wd1 = wd_v[pl.ds(16, 16)]

        z16 = jnp.zeros((16,), jnp.float32)

        def zrow(i, carry):
            zb_v[i, pl.ds(0, 16)] = z16
            zb_v[i, pl.ds(16, 16)] = z16
            return carry

        lax.fori_loop(0, ZR, zrow, 0)
        tile_row0 = s * rows_per_tile

        def zcopy(i, carry):
            pltpu.sync_copy(zb_v, agg_sh.at[pl.ds(tile_row0 + i * ZR, ZR)])
            return carry

        lax.fori_loop(0, nzc, zcopy, 0)
        plsc.subcore_barrier()

        ebase = s * per_tile

        def idx_copies(j, sl):
            src_v, dst_v = sl[0], sl[1]
            sem = sl[6]
            base = pl.multiple_of(ebase + j * C, 8)
            return (pltpu.make_async_copy(ei_h.at[pl.ds(base, C)], src_v, sem),
                    pltpu.make_async_copy(ei_h.at[pl.ds(E + base, C)], dst_v,
                                          sem))

        def gather_copies(sl):
            return (pltpu.make_async_copy(sa_h.at[sl[0]], sl[3], sl[7]),
                    pltpu.make_async_copy(sb_h.at[sl[2]], sl[4], sl[7]))

        def offset_ids(sl):
            src_v, dst_v, dstg_v = sl[0], sl[1], sl[2]
            for j in range(C // 16):
                w = pl.ds(j * 16, 16)
                src_v[w] = src_v[w] + cN
                dstg_v[w] = dst_v[w] + cN

        UE = 8

        def compute(sl):
            a_v, b_v, m_v = sl[3], sl[4], sl[5]

            def ebody(kk, carry):
                e0 = kk * UE
                for u in range(UE):
                    e = e0 + u
                    unp = functools.partial(
                        plsc.unpack, format=plsc.PackFormat.INTERLEAVED)
                    a0, a1 = unp(a_v[e, pl.ds(0, 32)])
                    b0, b1 = unp(b_v[e, pl.ds(0, 32)])
                    pe_a, po_a = unp(a_v[e, pl.ds(32, 32)])
                    pe_b, po_b = unp(b_v[e, pl.ds(32, 32)])
                    cpe = pe_a * pe_b
                    cpo = po_a * po_b
                    cr = cpe[0] + cpe[1] + cpo[0]
                    m0 = a0 + b0 + cr * wd0
                    m1 = a1 + b1 + cr * wd1
                    m_v[e, pl.ds(0, 16)] = _silu(m0)
                    m_v[e, pl.ds(16, 16)] = _silu(m1)
                return carry

            lax.fori_loop(0, C // UE, ebody, 0)

        def scatter_copy(sl):
            return pltpu.make_async_copy(sl[5], agg_sh.at[sl[9]], sl[8])

        for cp in idx_copies(0, slots[0]):
            cp.start()
        if nchunks > 1:
            for cp in idx_copies(1, slots[1]):
                cp.start()
        for cp in idx_copies(0, slots[0]):
            cp.wait()
        offset_ids(slots[0])
        for cp in gather_copies(slots[0]):
            cp.start()

        def step(i, p):
            sl = slots[p]
            sq = slots[1 - p]

            @pl.when(i < nchunks - 1)
            def _():
                for cp in idx_copies(i + 1, sq):
                    cp.wait()
                offset_ids(sq)
                for cp in gather_copies(sq):
                    cp.start()

            for cp in gather_copies(sl):
                cp.wait()

            @pl.when(i >= 2)
            def _():
                scatter_copy(sl).wait()

            dst_v, dsts_v = sl[1], sl[9]
            for j in range(C // 16):
                w = pl.ds(j * 16, 16)
                dsts_v[w] = dst_v[w]

            @pl.when(i < nchunks - 2)
            def _():
                for cp in idx_copies(i + 2, sl):
                    cp.start()

            compute(sl)
            scatter_copy(sl).start(add=True)

        def chunk(i, carry):
            @pl.when(i % 2 == 0)
            def _():
                step(i, 0)

            @pl.when(i % 2 == 1)
            def _():
                step(i, 1)

            return carry

        lax.fori_loop(0, nchunks, chunk, 0)
        if nchunks > 1:
            scatter_copy(slots[1]).wait()
        scatter_copy(slots[0]).wait()
        plsc.subcore_barrier()
        pltpu.sync_copy(agg_sh.at[pl.ds(tile_row0, rows_per_tile)],
                        out_h.at[c, pl.ds(tile_row0, rows_per_tile)])

    return k(ei.reshape(2 * E), SA, SB, wd2)


def _stage3_body(h_ref, agg_ref, wh_ref, bh_ref, wfc_ref, bfc_ref, out_ref):
    hv = h_ref[...]
    agg = agg_ref[...]
    Wh = wh_ref[...]
    t = (jnp.dot(hv, Wh[0:H], preferred_element_type=jnp.float32) +
         jnp.dot(agg, Wh[H:2 * H], preferred_element_type=jnp.float32) +
         bh_ref[...])
    hn = _silu(t)
    out_ref[...] = jnp.dot(hn, wfc_ref[...],
                           preferred_element_type=jnp.float32) + bfc_ref[...]


def _stage3(h, agg, W_h, b_h, W_fc, b_fc, interpret=False):
    N = h.shape[0]
    R = 400 if N % 400 == 0 else N
    grid = N // R
    full = lambda shape: pl.BlockSpec(shape, lambda i: (0, 0))
    row_blk = lambda w: pl.BlockSpec((R, w), lambda i: (i, 0))
    return pl.pallas_call(
        _stage3_body,
        grid=(grid,),
        in_specs=[row_blk(H), row_blk(H), full((2 * H, H)), full((1, H)),
                  full((H, 3)), full((1, 3))],
        out_specs=row_blk(3),
        out_shape=jax.ShapeDtypeStruct((N, 3), jnp.float32),
        interpret=interpret,
    )(h, agg, W_h, b_h.reshape(1, H), W_fc, b_fc.reshape(1, 3))


def kernel(pos, edge_index, W_embed, b_embed, W_e, b_e, W_h, b_h, W_fc, b_fc):
    N = pos.shape[0]
    E = edge_index.shape[1]
    h, sa0, sa1, sb0, sb1 = _stage1(pos, W_embed, b_embed, W_e, b_e)

    zt = jnp.zeros((N, 14), jnp.float32)
    pe = jnp.concatenate([pos[:, 0:1], pos[:, 2:3], zt], axis=1)
    po = jnp.concatenate([pos[:, 1:2], zt, zt[:, :1]], axis=1)

    def mkrow(F, t_even, t_odd):
        even = jnp.concatenate([F[:, :16], t_even], axis=1)
        odd = jnp.concatenate([F[:, 16:], t_odd], axis=1)
        return jnp.stack([even, odd], axis=-1).reshape(
            N, ROW).astype(jnp.bfloat16)

    SA = jnp.concatenate([mkrow(sa0, pe, po), mkrow(sa1, pe, po)], axis=0)
    SB = jnp.concatenate([mkrow(sb0, -2.0 * pe, -2.0 * po),
                          mkrow(sb1, -2.0 * pe, -2.0 * po)], axis=0)
    wd2 = W_e[2 * H].reshape(NC, HH)
    aggs = _sc_edge(edge_index, SA, SB, wd2, N, E)
    agg = jnp.concatenate([aggs[0, :N], aggs[1, :N]], axis=1)
    return _stage3(h, agg, W_h, b_h, W_fc, b_fc)
